# baseline (speedup 1.0000x reference)
import functools

import jax
import jax.numpy as jnp
from jax import lax
from jax.experimental import pallas as pl
from jax.experimental.pallas import tpu as pltpu
from jax.experimental.pallas import tpu_sc as plsc

_NB = 3
_LA = 2


def _make_sc_kernel(B, T, D):
    info = plsc.get_sparse_core_info()
    NC, NS, L = info.num_cores, info.num_subcores, info.num_lanes
    NW = NC * NS
    rows_per_w = T // NW
    CR = 8
    n_chunks = rows_per_w // CR
    n_col = D // L

    mesh = plsc.VectorSubcoreMesh(core_axis_name="c", subcore_axis_name="s")

    scratch = (
        [pltpu.VMEM((CR, D), jnp.float32) for _ in range(_NB * B)]
        + [pltpu.VMEM((CR, D), jnp.float32) for _ in range(_NB)]
        + [pltpu.SemaphoreType.DMA for _ in range(3 * _NB)]
    )

    @functools.partial(
        pl.kernel,
        mesh=mesh,
        out_type=jax.ShapeDtypeStruct((B * T, D), jnp.float32),
        scratch_types=scratch,
    )
    def k(x_hbm, pe_hbm, out_hbm, *refs):
        bufx = [refs[s * B:(s + 1) * B] for s in range(_NB)]
        bufp = refs[_NB * B:_NB * B + _NB]
        sem_lx = refs[_NB * B + _NB:_NB * B + 2 * _NB]
        sem_lp = refs[_NB * B + 2 * _NB:_NB * B + 3 * _NB]
        sem_st = refs[_NB * B + 3 * _NB:]

        wid = lax.axis_index("s") * NC + lax.axis_index("c")
        base = wid * rows_per_w

        def start_loads(i, s):
            t0 = base + i * CR
            for bb in range(B):
                pltpu.async_copy(x_hbm.at[pl.ds(bb * T + t0, CR)],
                                 bufx[s][bb], sem_lx[s])
            pltpu.async_copy(pe_hbm.at[pl.ds(t0, CR)], bufp[s], sem_lp[s])

        def wait_loads(i, s):
            t0 = base + i * CR
            for bb in range(B):
                pltpu.make_async_copy(x_hbm.at[pl.ds(bb * T + t0, CR)],
                                      bufx[s][bb], sem_lx[s]).wait()
            pltpu.make_async_copy(pe_hbm.at[pl.ds(t0, CR)], bufp[s],
                                  sem_lp[s]).wait()

        def wait_store(i, s):
            t0 = base + i * CR
            for bb in range(B):
                pltpu.make_async_copy(bufx[s][bb],
                                      out_hbm.at[pl.ds(bb * T + t0, CR)],
                                      sem_st[s]).wait()

        for s in range(_LA):
            start_loads(s, s)

        def iteration(i, s):
            wait_loads(i, s)

            G = 16

            @plsc.parallel_loop(0, CR, unroll=2)
            def _row(r):
                for g in range(n_col // G):
                    vs = [bufp[s][r, pl.ds((g * G + u) * L, L)]
                          for u in range(G)]
                    for bb in range(B):
                        for u in range(G):
                            plsc.addupdate(
                                bufx[s][bb].at[r, pl.ds((g * G + u) * L, L)],
                                vs[u])

            t0 = base + i * CR
            for bb in range(B):
                pltpu.async_copy(bufx[s][bb],
                                 out_hbm.at[pl.ds(bb * T + t0, CR)], sem_st[s])

            sn = (s + _LA) % _NB
            j = i + _LA

            if isinstance(i, int):
                if j < n_chunks:
                    if j - _NB >= 0:
                        wait_store(j - _NB, sn)
                    start_loads(j, sn)
            else:
                def prefetch(_):
                    lax.cond(i + _LA >= _NB,
                             lambda __: wait_store(j - _NB, sn),
                             lambda __: None, 0)
                    start_loads(j, sn)
                    return 0

                lax.cond(j < n_chunks, prefetch, lambda _: 0, 0)

        def group(g, carry):
            for s in range(_NB):
                iteration(g * _NB + s, s)
            return carry

        n_groups = n_chunks // _NB
        lax.fori_loop(0, n_groups, group, 0)
        for i in range(n_groups * _NB, n_chunks):
            iteration(i, i % _NB)

        for s in range(_NB):
            wait_store(n_chunks - _NB + s, (n_chunks - _NB + s) % _NB)

    return k


def kernel(x, pe_weight):
    B, T, D = x.shape
    k = _make_sc_kernel(B, T, D)
    out = k(x.reshape(B * T, D), pe_weight)
    return out.reshape(B, T, D)
